# baseline (speedup 1.0000x reference)
import functools

import jax
import jax.numpy as jnp
from jax import lax
from jax.experimental import pallas as pl
from jax.experimental.pallas import tpu as pltpu
from jax.experimental.pallas import tpu_sc as plsc

VOCAB = 1000000
DIM = 64
B = 4096
L = 200

NC = 2
NS = 16
NW = NC * NS
LANES = 16

BBLK = B // NW
STEPS = L
NBUF = 4
GROUPS = STEPS // NBUF


def _sc_gather(table, idxT):
    mesh = plsc.VectorSubcoreMesh(core_axis_name="c", subcore_axis_name="s")

    @functools.partial(
        pl.kernel,
        mesh=mesh,
        out_type=jax.ShapeDtypeStruct((L, DIM // 8, B // 128, 8, 128), jnp.float32),
        compiler_params=pltpu.CompilerParams(
            use_tc_tiling_on_sc=False, needs_layout_passes=False
        ),
        scratch_types=[
            pltpu.VMEM((STEPS, BBLK), jnp.int32),
        ]
        + [pltpu.VMEM((BBLK, DIM), jnp.float32) for _ in range(NBUF)]
        + [pltpu.VMEM((DIM // 8, 8, BBLK), jnp.float32) for _ in range(NBUF)]
        + [pltpu.SemaphoreType.DMA for _ in range(2 * NBUF)],
    )
    def k(table_hbm, idx_hbm, out_hbm, idx_v, *scratch):
        gb = list(scratch[:NBUF])
        tb = list(scratch[NBUF : 2 * NBUF])
        gsems = list(scratch[2 * NBUF : 3 * NBUF])
        wsems = list(scratch[3 * NBUF : 4 * NBUF])

        wid = lax.axis_index("s") * NC + lax.axis_index("c")

        pltpu.sync_copy(
            idx_hbm.at[pl.ds(0, STEPS), pl.ds(wid * BBLK, BBLK)], idx_v
        )

        iota = lax.iota(jnp.int32, LANES)
        rows_bg = [bg * LANES + iota for bg in range(BBLK // LANES)]

        def g_start(l, b):
            pltpu.async_copy(table_hbm.at[idx_v.at[l]], gb[b], gsems[b])

        def g_wait(l, b):
            pltpu.make_async_copy(table_hbm.at[idx_v.at[l]], gb[b], gsems[b]).wait()

        def transpose(b):
            def percol(c, carry):
                col = jnp.full((LANES,), c, jnp.int32)
                vals = [
                    plsc.load_gather(gb[b], [rows_bg[bg], col])
                    for bg in range(BBLK // LANES)
                ]
                for bg in range(BBLK // LANES):
                    tb[b][c >> 3, c & 7, pl.ds(bg * LANES, LANES)] = vals[bg]
                return carry

            lax.fori_loop(0, DIM, percol, 0)

        def w_start(l, b):
            pltpu.async_copy(tb[b], out_hbm.at[l, pl.ds(0, DIM // 8), wid], wsems[b])

        def w_wait(l, b):
            pltpu.make_async_copy(
                tb[b], out_hbm.at[l, pl.ds(0, DIM // 8), wid], wsems[b]
            ).wait()

        for b in range(NBUF):
            g_start(b, b)

        def group(g, carry):
            l0 = g * NBUF
            for b in range(NBUF):
                g_wait(l0 + b, b)
                transpose(b)
                w_start(l0 + b, b)
            for b in range(NBUF):
                w_wait(l0 + b, b)
                nxt = l0 + NBUF + b

                @pl.when(nxt < STEPS)
                def _():
                    g_start(nxt, b)

            return carry

        lax.fori_loop(0, GROUPS, group, 0)

    return k(table, idxT)


def kernel(X_word, X_char, word_embed):
    del X_char
    idxT = X_word.T
    out5d = _sc_gather(word_embed, idxT)
    return out5d.transpose(2, 4, 0, 1, 3).reshape(B, L, DIM)


# speedup vs baseline: 1.7108x; 1.7108x over previous
import functools

import jax
import jax.numpy as jnp
from jax import lax
from jax.experimental import pallas as pl
from jax.experimental.pallas import tpu as pltpu
from jax.experimental.pallas import tpu_sc as plsc

VOCAB = 1000000
DIM = 64
B = 4096
L = 200

NC = 2
NS = 16
NW = NC * NS
LANES = 16

BBLK = B // NW
STEPS = L
NBUF = 4
GROUPS = STEPS // NBUF


def _sc_gather(table, idxT):
    mesh = plsc.VectorSubcoreMesh(core_axis_name="c", subcore_axis_name="s")

    @functools.partial(
        pl.kernel,
        mesh=mesh,
        out_type=jax.ShapeDtypeStruct((L, DIM // 8, B // 128, 8, 128), jnp.float32),
        compiler_params=pltpu.CompilerParams(
            use_tc_tiling_on_sc=False, needs_layout_passes=False
        ),
        scratch_types=[
            pltpu.VMEM((STEPS, BBLK), jnp.int32),
        ]
        + [pltpu.VMEM((BBLK, DIM), jnp.float32) for _ in range(NBUF)]
        + [pltpu.VMEM((DIM // 8, 8, BBLK + 1), jnp.float32) for _ in range(NBUF)]
        + [pltpu.SemaphoreType.DMA for _ in range(2 * NBUF)],
    )
    def k(table_hbm, idx_hbm, out_hbm, idx_v, *scratch):
        gb = list(scratch[:NBUF])
        tb = list(scratch[NBUF : 2 * NBUF])
        gsems = list(scratch[2 * NBUF : 3 * NBUF])
        wsems = list(scratch[3 * NBUF : 4 * NBUF])

        wid = lax.axis_index("s") * NC + lax.axis_index("c")

        pltpu.sync_copy(
            idx_hbm.at[pl.ds(0, STEPS), pl.ds(wid * BBLK, BBLK)], idx_v
        )

        iota = lax.iota(jnp.int32, LANES)
        d0 = [(cg * LANES + iota) >> 3 for cg in range(DIM // LANES)]
        d1 = [(cg * LANES + iota) & 7 for cg in range(DIM // LANES)]

        def g_start(l, b):
            pltpu.async_copy(table_hbm.at[idx_v.at[l]], gb[b], gsems[b])

        def g_wait(l, b):
            pltpu.make_async_copy(table_hbm.at[idx_v.at[l]], gb[b], gsems[b]).wait()

        def transpose(b):
            def perrow(b0, carry):
                vb0 = jnp.full((LANES,), b0, jnp.int32)
                vals = [
                    gb[b][b0, pl.ds(cg * LANES, LANES)]
                    for cg in range(DIM // LANES)
                ]
                for cg in range(DIM // LANES):
                    plsc.store_scatter(tb[b], [d0[cg], d1[cg], vb0], vals[cg])
                return carry

            lax.fori_loop(0, BBLK, perrow, 0)

        def w_start(l, b):
            pltpu.async_copy(
                tb[b].at[pl.ds(0, DIM // 8), pl.ds(0, 8), pl.ds(0, BBLK)],
                out_hbm.at[l, pl.ds(0, DIM // 8), wid],
                wsems[b],
            )

        def w_wait(l, b):
            pltpu.make_async_copy(
                tb[b].at[pl.ds(0, DIM // 8), pl.ds(0, 8), pl.ds(0, BBLK)],
                out_hbm.at[l, pl.ds(0, DIM // 8), wid],
                wsems[b],
            ).wait()

        for b in range(NBUF):
            g_start(b, b)

        def group(g, carry):
            l0 = g * NBUF
            for b in range(NBUF):
                g_wait(l0 + b, b)
                transpose(b)
                w_start(l0 + b, b)
            for b in range(NBUF):
                w_wait(l0 + b, b)
                nxt = l0 + NBUF + b

                @pl.when(nxt < STEPS)
                def _():
                    g_start(nxt, b)

            return carry

        lax.fori_loop(0, GROUPS, group, 0)

    return k(table, idxT)


def kernel(X_word, X_char, word_embed):
    del X_char
    idxT = X_word.T
    out5d = _sc_gather(word_embed, idxT)
    return out5d.transpose(2, 4, 0, 1, 3).reshape(B, L, DIM)
